# Initial kernel scaffold; baseline (speedup 1.0000x reference)
import jax
import jax.numpy as jnp
from jax.experimental import pallas as pl


def kernel(proj_range, unproj_range, proj_argmax, px, py, pz):
    raise NotImplementedError("write your pallas kernel here")


# speedup vs baseline: 33.2872x; 33.2872x over previous
import functools

import jax
import jax.numpy as jnp
from jax import lax
from jax.experimental import pallas as pl
from jax.experimental.pallas import tpu as pltpu
from jax.experimental.pallas import tpu_sc as plsc

D, H, W = 16, 64, 256
P = 131072
Dp, Hp, Wp = D + 4, H + 4, W + 4
SZ = Hp * Wp
G = Dp * Hp * Wp
NC, NS, L = 2, 16, 16
NW = NC * NS
PW = P // NW
CH = 128
NCH = PW // CH
NG = CH // L
S3 = 125
CENTER = 62
CUTOFF = jnp.float32(1.0)
NCLASSES = 21
INF = jnp.float32(jnp.inf)

_mesh = plsc.VectorSubcoreMesh(core_axis_name="c", subcore_axis_name="s")


@functools.partial(
    pl.kernel,
    out_type=jax.ShapeDtypeStruct((P,), jnp.int32),
    mesh=_mesh,
    scratch_types=[
        pltpu.VMEM((PW,), jnp.int32),
        pltpu.VMEM((PW,), jnp.int32),
        pltpu.VMEM((PW,), jnp.int32),
        pltpu.VMEM((PW,), jnp.float32),
        pltpu.VMEM((PW,), jnp.int32),
        pltpu.VMEM((S3 * CH,), jnp.int32),
        pltpu.VMEM((S3 * CH,), jnp.float32),
        pltpu.VMEM((S3 * CH,), jnp.int32),
    ],
)
def _bev_knn(rg_hbm, cg_hbm, unp_hbm, px_hbm, py_hbm, pz_hbm, out_hbm,
             px_v, py_v, base_v, unp_v, out_v, idx_v, rv_v, cv_v):
    cid = lax.axis_index("c")
    sid = lax.axis_index("s")
    wid = sid * NC + cid
    start = wid * PW

    pltpu.sync_copy(px_hbm.at[pl.ds(start, PW)], px_v)
    pltpu.sync_copy(py_hbm.at[pl.ds(start, PW)], py_v)
    pltpu.sync_copy(pz_hbm.at[pl.ds(start, PW)], base_v)
    pltpu.sync_copy(unp_hbm.at[pl.ds(start, PW)], unp_v)

    def base_body(g, _):
        sl = pl.ds(g * L, L)
        base_v[sl] = base_v[sl] * SZ + py_v[sl] * Wp + px_v[sl]
        return 0

    lax.fori_loop(0, PW // L, base_body, 0)

    def chunk_body(ci, _):
        p0 = ci * CH

        def build_body(srow, _):
            dz = srow // 25
            rem = srow - dz * 25
            dy = rem // 5
            dx = rem - dy * 5
            off = dz * SZ + dy * Wp + dx
            for g in range(NG):
                idx_v[pl.ds(srow * CH + g * L, L)] = (
                    base_v[pl.ds(p0 + g * L, L)] + off)
            return 0

        lax.fori_loop(0, S3, build_body, 0)

        pltpu.sync_copy(rg_hbm.at[idx_v], rv_v)
        pltpu.sync_copy(cg_hbm.at[idx_v], cv_v)

        def center_body(g, _):
            rv_v[pl.ds(CENTER * CH + g * L, L)] = unp_v[pl.ds(p0 + g * L, L)]
            return 0

        lax.fori_loop(0, NG, center_body, 0)

        def group_body_bisect(g, _):
            out_v[pl.ds(p0 + g * L, L)] = cv_v[pl.ds(CENTER * CH + g * L, L)]
            return 0

        def group_body(g, _):
            lane = pl.ds(g * L, L)
            unp = unp_v[pl.ds(p0 + g * L, L)]
            zero = jnp.zeros((L,), jnp.float32)
            init = (zero + INF, zero + INF, zero + INF, zero + INF, zero + INF,
                    jnp.zeros((L,), jnp.int32), jnp.zeros((L,), jnp.int32),
                    jnp.zeros((L,), jnp.int32), jnp.zeros((L,), jnp.int32),
                    jnp.zeros((L,), jnp.int32))

            def s_body(s, carry):
                bd0, bd1, bd2, bd3, bd4, bc0, bc1, bc2, bc3, bc4 = carry
                rv = rv_v[pl.ds(s * CH + g * L, L)]
                cv = cv_v[pl.ds(s * CH + g * L, L)]
                d = jnp.abs(rv - unp)
                d = jnp.where(rv < jnp.float32(0.0), INF, d)
                m0 = d < bd0
                m1 = d < bd1
                m2 = d < bd2
                m3 = d < bd3
                m4 = d < bd4
                nd0 = jnp.where(m0, d, bd0)
                nc0 = jnp.where(m0, cv, bc0)
                nd1 = jnp.where(m1, jnp.where(m0, bd0, d), bd1)
                nc1 = jnp.where(m1, jnp.where(m0, bc0, cv), bc1)
                nd2 = jnp.where(m2, jnp.where(m1, bd1, d), bd2)
                nc2 = jnp.where(m2, jnp.where(m1, bc1, cv), bc2)
                nd3 = jnp.where(m3, jnp.where(m2, bd2, d), bd3)
                nc3 = jnp.where(m3, jnp.where(m2, bc2, cv), bc3)
                nd4 = jnp.where(m4, jnp.where(m3, bd3, d), bd4)
                nc4 = jnp.where(m4, jnp.where(m3, bc3, cv), bc4)
                return (nd0, nd1, nd2, nd3, nd4, nc0, nc1, nc2, nc3, nc4)

            bd0, bd1, bd2, bd3, bd4, bc0, bc1, bc2, bc3, bc4 = lax.fori_loop(
                0, S3, s_body, init)

            cls = [
                jnp.where(bd0 > CUTOFF, NCLASSES, bc0),
                jnp.where(bd1 > CUTOFF, NCLASSES, bc1),
                jnp.where(bd2 > CUTOFF, NCLASSES, bc2),
                jnp.where(bd3 > CUTOFF, NCLASSES, bc3),
                jnp.where(bd4 > CUTOFF, NCLASSES, bc4),
            ]
            one = jnp.ones((L,), jnp.int32)
            nil = jnp.zeros((L,), jnp.int32)
            best_s = jnp.zeros((L,), jnp.int32)
            best_c = jnp.ones((L,), jnp.int32)
            for j in range(5):
                cnt = jnp.zeros((L,), jnp.int32)
                for k in range(5):
                    cnt = cnt + jnp.where(cls[j] == cls[k], one, nil)
                valid = (cls[j] >= 1) & (cls[j] <= 20)
                score = jnp.where(valid, cnt * 32 + (31 - cls[j]), nil)
                better = score > best_s
                best_s = jnp.where(better, score, best_s)
                best_c = jnp.where(better, cls[j], best_c)
            out_v[pl.ds(p0 + g * L, L)] = best_c
            return 0

        lax.fori_loop(0, NG, group_body, 0)
        return 0

    lax.fori_loop(0, NCH, chunk_body, 0)
    pltpu.sync_copy(out_v, out_hbm.at[pl.ds(start, PW)])


def kernel(proj_range, unproj_range, proj_argmax, px, py, pz):
    rg = jnp.pad(proj_range, ((2, 2), (2, 2), (2, 2))).reshape(-1)
    cg = jnp.pad(proj_argmax.astype(jnp.int32), ((2, 2), (2, 2), (2, 2))).reshape(-1)
    return _bev_knn(rg, cg, unproj_range,
                    px.astype(jnp.int32), py.astype(jnp.int32),
                    pz.astype(jnp.int32))
